# Initial kernel scaffold; baseline (speedup 1.0000x reference)
import jax
import jax.numpy as jnp
from jax.experimental import pallas as pl


def kernel(x, edge_index, sim_prompt, deg_prompt, other_prompt, in_proj_w, in_proj_b, out_proj_w, out_proj_b):
    raise NotImplementedError("write your pallas kernel here")


# speedup vs baseline: 4.4106x; 4.4106x over previous
import functools

import jax
import jax.numpy as jnp
import numpy as np
from jax import lax
from jax.experimental import pallas as pl
from jax.experimental.pallas import tpu as pltpu
from jax.experimental.pallas import tpu_sc as plsc

N = 10000
C = 256
E = 160000

NW = 32
EPT = E // NW
CH = 32
RING = 4
NCH = (EPT + CH - 1) // CH
PADDED = NCH * CH
LANES = 16
BSTEPS = C // 32


def _normalize_body(x_ref, o_ref):
    xb = x_ref[...]
    xn = xb / jnp.sqrt(jnp.sum(xb * xb, axis=1, keepdims=True))
    o_ref[...] = xn.astype(jnp.bfloat16)


def _normalize(x):
    nb = 1000
    return pl.pallas_call(
        _normalize_body,
        grid=(N // nb,),
        in_specs=[pl.BlockSpec((nb, C), lambda i: (i, 0))],
        out_specs=pl.BlockSpec((nb, C), lambda i: (i, 0)),
        out_shape=jax.ShapeDtypeStruct((N, C), jnp.bfloat16),
    )(x)


@functools.cache
def _edge_accum_fn():
    mesh = plsc.VectorSubcoreMesh(core_axis_name="c", subcore_axis_name="s")
    return functools.partial(
        pl.kernel,
        mesh=mesh,
        compiler_params=pltpu.CompilerParams(needs_layout_passes=False),
        out_type=[
            jax.ShapeDtypeStruct((NW, N), jnp.float32),
            jax.ShapeDtypeStruct((NW, N), jnp.float32),
        ],
        scratch_types=[
            pltpu.VMEM((PADDED,), jnp.int32),
            pltpu.VMEM((PADDED,), jnp.int32),
            pltpu.VMEM((RING, CH, C // 2), jnp.int32),
            pltpu.VMEM((RING, CH, C // 2), jnp.int32),
            pltpu.VMEM((N,), jnp.float32),
            pltpu.VMEM((N,), jnp.float32),
            pltpu.SemaphoreType.DMA,
            pltpu.SemaphoreType.DMA,
            pltpu.SemaphoreType.DMA,
            pltpu.SemaphoreType.DMA,
        ],
    )(_edge_accum_body)


def _edge_accum_body(xn_hbm, row_hbm, col_hbm, c_out, d_out,
                     row_v, col_v, abuf, bbuf, cacc, dacc,
                     sem0, sem1, sem2, sem3):
    sems = (sem0, sem1, sem2, sem3)
    wid = lax.axis_index("s") * 2 + lax.axis_index("c")
    base = wid * EPT

    zero16f = jnp.zeros((LANES,), jnp.float32)
    zero16i = jnp.zeros((LANES,), jnp.int32)
    ones16f = jnp.ones((LANES,), jnp.float32)
    lanes = lax.iota(jnp.int32, LANES)

    for k in range((PADDED - (EPT // LANES) * LANES) // LANES + 1):
        row_v[pl.ds(PADDED - (k + 1) * LANES, LANES)] = zero16i
        col_v[pl.ds(PADDED - (k + 1) * LANES, LANES)] = zero16i
    pltpu.sync_copy(row_hbm.at[pl.ds(base, EPT)], row_v.at[pl.ds(0, EPT)])
    pltpu.sync_copy(col_hbm.at[pl.ds(base, EPT)], col_v.at[pl.ds(0, EPT)])

    def _start(ch, slot):
        rowi = row_v.at[pl.ds(ch * CH, CH)]
        coli = col_v.at[pl.ds(ch * CH, CH)]
        pltpu.async_copy(xn_hbm.at[rowi], abuf.at[slot], sems[slot])
        pltpu.async_copy(xn_hbm.at[coli], bbuf.at[slot], sems[slot])

    def _wait(slot):
        pltpu.make_async_copy(
            xn_hbm.at[pl.ds(0, CH)], abuf.at[slot], sems[slot]).wait()
        pltpu.make_async_copy(
            xn_hbm.at[pl.ds(0, CH)], bbuf.at[slot], sems[slot]).wait()

    dnums = lax.GatherDimensionNumbers(
        offset_dims=(), collapsed_slice_dims=(0,), start_index_map=(0,))
    def _perm_xor(v, sh):
        return lax.gather(v, (lanes ^ sh)[:, None], dnums, slice_sizes=(1,),
                          mode=lax.GatherScatterMode.PROMISE_IN_BOUNDS)

    def _compute(ch, slot):
        for g in range(CH // LANES):
            stack = []
            for e_ in range(LANES):
                def _ld(buf, t):
                    v = buf[slot, g * LANES + e_, pl.ds(t * LANES, LANES)]
                    return plsc.bitcast(v, jnp.bfloat16)

                acc = _ld(abuf, 0) * _ld(bbuf, 0)
                for t in range(1, BSTEPS):
                    acc = acc + _ld(abuf, t) * _ld(bbuf, t)
                u0, u1 = plsc.unpack(acc, format=plsc.PackFormat.INTERLEAVED,
                                     preferred_element_type=jnp.float32)
                f = u0 + u1
                lvl = 0
                while stack and stack[-1][0] == lvl:
                    s = 1 << lvl
                    a = stack.pop()[1]
                    a = a + _perm_xor(a, s)
                    b = f + _perm_xor(f, s)
                    f = jnp.where((lanes & s) != 0, b, a)
                    lvl += 1
                stack.append((lvl, f))
            dvec = stack[0][1]
            off = ch * CH + g * LANES
            coli = col_v[pl.ds(off, LANES)]
            valid = (off + lanes) < EPT
            plsc.addupdate_scatter(cacc, [coli], dvec, mask=valid)
            plsc.addupdate_scatter(dacc, [coli], ones16f, mask=valid)

    for p in range(RING - 1):
        _start(p, p)

    def _zbody(i, carry):
        cacc[pl.ds(i * LANES, LANES)] = zero16f
        dacc[pl.ds(i * LANES, LANES)] = zero16f
        return carry
    lax.fori_loop(0, N // LANES, _zbody, 0)

    def _body(i, carry):
        for b in range(RING):
            ch = i * RING + b

            @pl.when(ch < NCH)
            def _():
                @pl.when(ch + RING - 1 < NCH)
                def _():
                    _start(ch + RING - 1, (b + RING - 1) % RING)
                _wait(b)
                _compute(ch, b)
        return carry

    lax.fori_loop(0, (NCH + RING - 1) // RING, _body, 0)

    pltpu.sync_copy(cacc, c_out.at[wid])
    pltpu.sync_copy(dacc, d_out.at[wid])


def _table(simp_ref, degp_ref, othp_ref, w_ref, b_ref, wo_ref, bo_ref):
    w = w_ref[...]
    bias = b_ref[...]
    wo = wo_ref[...]
    bo = bo_ref[...]
    scale = 1.0 / float(np.sqrt(C))
    neg1 = jnp.full((1, C), -1.0, jnp.float32)
    rows = []
    for case in range(4):
        s_ = bool(case & 1)
        d_ = bool(case & 2)
        o_ = (not s_) and (not d_)
        slot0 = simp_ref[...] if s_ else neg1
        slot1 = degp_ref[...] if d_ else neg1
        slot2 = othp_ref[...] if o_ else neg1
        rec = jnp.concatenate([slot0, slot1, slot2], axis=0)
        qkv = lax.dot_general(rec, w, (((1,), (1,)), ((), ()))) + bias
        q = qkv[:, :C]
        k = qkv[:, C:2 * C]
        v = qkv[:, 2 * C:]
        attn = lax.dot_general(q, k, (((1,), (1,)), ((), ()))) * scale
        colid = lax.broadcasted_iota(jnp.int32, (3, 3), 1)
        for qi, valid in enumerate((s_, d_, o_)):
            if not valid:
                attn = jnp.where(colid == qi, jnp.float32(-1e30), attn)
        attn = jax.nn.softmax(attn, axis=-1)
        out = jnp.dot(attn, v)
        out = lax.dot_general(out, wo, (((1,), (1,)), ((), ()))) + bo
        rows.append(jnp.mean(out, axis=0, keepdims=True))
    return jnp.concatenate(rows, axis=0)


def _combine_body(cp_ref, dp_ref, x_ref, simp_ref, degp_ref, othp_ref,
                  w_ref, b_ref, wo_ref, bo_ref, o_ref, t_ref):
    @pl.when(pl.program_id(0) == 0)
    def _():
        t_ref[...] = _table(simp_ref, degp_ref, othp_ref,
                            w_ref, b_ref, wo_ref, bo_ref)

    c = jnp.sum(cp_ref[...], axis=1, keepdims=True)
    deg = jnp.sum(dp_ref[...], axis=1, keepdims=True)
    csim = c / deg
    mask_sim = csim <= 0.6
    mask_deg = deg <= 2.0
    case = mask_sim.astype(jnp.int32) + 2 * mask_deg.astype(jnp.int32)
    nb = cp_ref.shape[0]
    oh = (case == lax.broadcasted_iota(jnp.int32, (nb, 4), 1))
    o_ref[...] = x_ref[...] + jnp.dot(oh.astype(jnp.float32), t_ref[...])


def _combine(cp_t, dp_t, x, simp, degp, othp, w, b2, wo, bo2):
    nb = 1000
    full = lambda shape: pl.BlockSpec(shape, lambda i: tuple(0 for _ in shape))
    return pl.pallas_call(
        _combine_body,
        grid=(N // nb,),
        in_specs=[
            pl.BlockSpec((nb, NW), lambda i: (i, 0)),
            pl.BlockSpec((nb, NW), lambda i: (i, 0)),
            pl.BlockSpec((nb, C), lambda i: (i, 0)),
            full((1, C)), full((1, C)), full((1, C)),
            full((3 * C, C)), full((1, 3 * C)),
            full((C, C)), full((1, C)),
        ],
        out_specs=pl.BlockSpec((nb, C), lambda i: (i, 0)),
        out_shape=jax.ShapeDtypeStruct((N, C), jnp.float32),
        scratch_shapes=[pltpu.VMEM((4, C), jnp.float32)],
    )(cp_t, dp_t, x, simp, degp, othp, w, b2, wo, bo2)


def kernel(x, edge_index, sim_prompt, deg_prompt, other_prompt,
           in_proj_w, in_proj_b, out_proj_w, out_proj_b):
    xn = _normalize(x)
    xn_i32 = lax.bitcast_convert_type(xn.reshape(N, C // 2, 2), jnp.int32)
    c_parts, d_parts = _edge_accum_fn()(xn_i32, edge_index[0], edge_index[1])
    return _combine(c_parts.T, d_parts.T, x,
                    sim_prompt, deg_prompt, other_prompt,
                    in_proj_w, in_proj_b.reshape(1, -1),
                    out_proj_w, out_proj_b.reshape(1, -1))


# speedup vs baseline: 4.4583x; 1.0108x over previous
import functools

import jax
import jax.numpy as jnp
import numpy as np
from jax import lax
from jax.experimental import pallas as pl
from jax.experimental.pallas import tpu as pltpu
from jax.experimental.pallas import tpu_sc as plsc

N = 10000
C = 256
E = 160000

NW = 32
EPT = E // NW
CH = 32
RING = 4
NCH = (EPT + CH - 1) // CH
PADDED = NCH * CH
LANES = 16
BSTEPS = C // 32


def _normalize_body(x_ref, o_ref):
    xb = x_ref[...]
    xn = xb / jnp.sqrt(jnp.sum(xb * xb, axis=1, keepdims=True))
    o_ref[...] = xn.astype(jnp.bfloat16)


def _normalize(x):
    nb = 1000
    return pl.pallas_call(
        _normalize_body,
        grid=(N // nb,),
        in_specs=[pl.BlockSpec((nb, C), lambda i: (i, 0))],
        out_specs=pl.BlockSpec((nb, C), lambda i: (i, 0)),
        out_shape=jax.ShapeDtypeStruct((N, C), jnp.bfloat16),
    )(x)


@functools.cache
def _edge_accum_fn():
    mesh = plsc.VectorSubcoreMesh(core_axis_name="c", subcore_axis_name="s")
    return functools.partial(
        pl.kernel,
        mesh=mesh,
        compiler_params=pltpu.CompilerParams(needs_layout_passes=False),
        out_type=[
            jax.ShapeDtypeStruct((NW, N), jnp.float32),
            jax.ShapeDtypeStruct((NW, N), jnp.float32),
        ],
        scratch_types=[
            pltpu.VMEM((PADDED,), jnp.int32),
            pltpu.VMEM((PADDED,), jnp.int32),
            pltpu.VMEM((RING, CH, C // 2), jnp.int32),
            pltpu.VMEM((RING, CH, C // 2), jnp.int32),
            pltpu.VMEM((N,), jnp.float32),
            pltpu.VMEM((N,), jnp.float32),
            pltpu.SemaphoreType.DMA,
            pltpu.SemaphoreType.DMA,
            pltpu.SemaphoreType.DMA,
            pltpu.SemaphoreType.DMA,
        ],
    )(_edge_accum_body)


def _edge_accum_body(xn_hbm, row_hbm, col_hbm, c_out, d_out,
                     row_v, col_v, abuf, bbuf, cacc, dacc,
                     sem0, sem1, sem2, sem3):
    sems = (sem0, sem1, sem2, sem3)
    wid = lax.axis_index("s") * 2 + lax.axis_index("c")
    base = wid * EPT

    zero16f = jnp.zeros((LANES,), jnp.float32)
    zero16i = jnp.zeros((LANES,), jnp.int32)
    ones16f = jnp.ones((LANES,), jnp.float32)
    lanes = lax.iota(jnp.int32, LANES)

    for k in range((PADDED - (EPT // LANES) * LANES) // LANES + 1):
        row_v[pl.ds(PADDED - (k + 1) * LANES, LANES)] = zero16i
        col_v[pl.ds(PADDED - (k + 1) * LANES, LANES)] = zero16i
    pltpu.sync_copy(row_hbm.at[pl.ds(base, EPT)], row_v.at[pl.ds(0, EPT)])
    pltpu.sync_copy(col_hbm.at[pl.ds(base, EPT)], col_v.at[pl.ds(0, EPT)])

    def _start(ch, slot):
        rowi = row_v.at[pl.ds(ch * CH, CH)]
        coli = col_v.at[pl.ds(ch * CH, CH)]
        pltpu.async_copy(xn_hbm.at[rowi], abuf.at[slot], sems[slot])
        pltpu.async_copy(xn_hbm.at[coli], bbuf.at[slot], sems[slot])

    def _wait(slot):
        pltpu.make_async_copy(
            xn_hbm.at[pl.ds(0, CH)], abuf.at[slot], sems[slot]).wait()
        pltpu.make_async_copy(
            xn_hbm.at[pl.ds(0, CH)], bbuf.at[slot], sems[slot]).wait()

    dnums = lax.GatherDimensionNumbers(
        offset_dims=(), collapsed_slice_dims=(0,), start_index_map=(0,))

    def _perm_xor(v, sh):
        return lax.gather(v, (lanes ^ sh)[:, None], dnums, slice_sizes=(1,),
                          mode=lax.GatherScatterMode.PROMISE_IN_BOUNDS)

    def _compute(ch, slot):
        for g in range(CH // LANES):
            stack = []
            for e_ in range(LANES):
                def _ld(buf, t):
                    v = buf[slot, g * LANES + e_, pl.ds(t * LANES, LANES)]
                    return plsc.bitcast(v, jnp.bfloat16)

                acc = _ld(abuf, 0) * _ld(bbuf, 0)
                for t in range(1, BSTEPS):
                    acc = acc + _ld(abuf, t) * _ld(bbuf, t)
                u0, u1 = plsc.unpack(acc, format=plsc.PackFormat.INTERLEAVED,
                                     preferred_element_type=jnp.float32)
                f = u0 + u1
                lvl = 0
                while stack and stack[-1][0] == lvl:
                    s = 1 << lvl
                    a = stack.pop()[1]
                    a = a + _perm_xor(a, s)
                    b = f + _perm_xor(f, s)
                    f = jnp.where((lanes & s) != 0, b, a)
                    lvl += 1
                stack.append((lvl, f))
            dvec = stack[0][1]
            off = ch * CH + g * LANES
            coli = col_v[pl.ds(off, LANES)]
            valid = (off + lanes) < EPT
            plsc.addupdate_scatter(cacc, [coli], dvec, mask=valid)
            plsc.addupdate_scatter(dacc, [coli], ones16f, mask=valid)

    for p in range(RING - 1):
        _start(p, p)

    def _zbody(i, carry):
        cacc[pl.ds(i * LANES, LANES)] = zero16f
        dacc[pl.ds(i * LANES, LANES)] = zero16f
        return carry
    lax.fori_loop(0, N // LANES, _zbody, 0)

    def _body(i, carry):
        for b in range(RING):
            ch = i * RING + b

            @pl.when(ch < NCH)
            def _():
                @pl.when(ch + RING - 1 < NCH)
                def _():
                    _start(ch + RING - 1, (b + RING - 1) % RING)
                _wait(b)
                _compute(ch, b)
        return carry

    lax.fori_loop(0, (NCH + RING - 1) // RING, _body, 0)

    pltpu.sync_copy(cacc, c_out.at[wid])
    pltpu.sync_copy(dacc, d_out.at[wid])


def _table_body(simp_ref, degp_ref, othp_ref, w_ref, b_ref, wo_ref, bo_ref,
                o_ref):
    w = w_ref[...]
    bias = b_ref[...]
    wo = wo_ref[...]
    bo = bo_ref[...]
    scale = 1.0 / float(np.sqrt(C))
    neg1 = jnp.full((1, C), -1.0, jnp.float32)
    rows = []
    for case in range(4):
        s_ = bool(case & 1)
        d_ = bool(case & 2)
        o_ = (not s_) and (not d_)
        slot0 = simp_ref[...] if s_ else neg1
        slot1 = degp_ref[...] if d_ else neg1
        slot2 = othp_ref[...] if o_ else neg1
        rec = jnp.concatenate([slot0, slot1, slot2], axis=0)
        qkv = lax.dot_general(rec, w, (((1,), (1,)), ((), ()))) + bias
        q = qkv[:, :C]
        k = qkv[:, C:2 * C]
        v = qkv[:, 2 * C:]
        attn = lax.dot_general(q, k, (((1,), (1,)), ((), ()))) * scale
        colid = lax.broadcasted_iota(jnp.int32, (3, 3), 1)
        for qi, valid in enumerate((s_, d_, o_)):
            if not valid:
                attn = jnp.where(colid == qi, jnp.float32(-1e30), attn)
        attn = jax.nn.softmax(attn, axis=-1)
        out = jnp.dot(attn, v)
        out = lax.dot_general(out, wo, (((1,), (1,)), ((), ()))) + bo
        rows.append(jnp.mean(out, axis=0, keepdims=True))
    o_ref[...] = jnp.concatenate(rows, axis=0)


def _attn_table(simp, degp, othp, w, b2, wo, bo2):
    full = lambda shape: pl.BlockSpec(shape, lambda: tuple(0 for _ in shape))
    return pl.pallas_call(
        _table_body,
        in_specs=[full((1, C)), full((1, C)), full((1, C)),
                  full((3 * C, C)), full((1, 3 * C)),
                  full((C, C)), full((1, C))],
        out_specs=full((4, C)),
        out_shape=jax.ShapeDtypeStruct((4, C), jnp.float32),
    )(simp, degp, othp, w, b2, wo, bo2)


def _combine_body(cp_ref, dp_ref, x_ref, t_ref, o_ref):
    c = jnp.sum(cp_ref[...], axis=1, keepdims=True)
    deg = jnp.sum(dp_ref[...], axis=1, keepdims=True)
    csim = c / deg
    mask_sim = csim <= 0.6
    mask_deg = deg <= 2.0
    case = mask_sim.astype(jnp.int32) + 2 * mask_deg.astype(jnp.int32)
    nb = cp_ref.shape[0]
    oh = (case == lax.broadcasted_iota(jnp.int32, (nb, 4), 1))
    o_ref[...] = x_ref[...] + jnp.dot(oh.astype(jnp.float32), t_ref[...])


def _combine(cp_t, dp_t, x, table):
    nb = 1000
    return pl.pallas_call(
        _combine_body,
        grid=(N // nb,),
        in_specs=[
            pl.BlockSpec((nb, NW), lambda i: (i, 0)),
            pl.BlockSpec((nb, NW), lambda i: (i, 0)),
            pl.BlockSpec((nb, C), lambda i: (i, 0)),
            pl.BlockSpec((4, C), lambda i: (0, 0)),
        ],
        out_specs=pl.BlockSpec((nb, C), lambda i: (i, 0)),
        out_shape=jax.ShapeDtypeStruct((N, C), jnp.float32),
    )(cp_t, dp_t, x, table)


def kernel(x, edge_index, sim_prompt, deg_prompt, other_prompt,
           in_proj_w, in_proj_b, out_proj_w, out_proj_b):
    xn = _normalize(x)
    xn_i32 = lax.bitcast_convert_type(xn.reshape(N, C // 2, 2), jnp.int32)
    c_parts, d_parts = _edge_accum_fn()(xn_i32, edge_index[0], edge_index[1])
    table = _attn_table(sim_prompt, deg_prompt, other_prompt,
                        in_proj_w, in_proj_b.reshape(1, -1),
                        out_proj_w, out_proj_b.reshape(1, -1))
    return _combine(c_parts.T, d_parts.T, x, table)
